# baseline (speedup 1.0000x reference)
import functools

import jax
import jax.numpy as jnp
from jax import lax
from jax.experimental import pallas as pl
from jax.experimental.pallas import tpu as pltpu
from jax.experimental.pallas import tpu_sc as plsc

VOCAB = 100000
D = 64
B, S = 1024, 200
BBLK = 128
UNITS = S * (B // BBLK)
PAIRS = 16
PER_PAIR = UNITS // PAIRS
Q0 = 24
Q1 = PER_PAIR - Q0
QMAX = max(Q0, Q1)
NBUF = 4

_mesh = plsc.VectorSubcoreMesh(core_axis_name="c", subcore_axis_name="s")


@functools.partial(
    pl.kernel,
    mesh=_mesh,
    out_type=jax.ShapeDtypeStruct((S, D // 8, B // BBLK, 8, BBLK), jnp.float32),
    scratch_types=[
        pltpu.VMEM((QMAX, BBLK), jnp.int32),
        pltpu.VMEM((S, D), jnp.float32),
        pltpu.VMEM((BBLK, D), jnp.float32),
        pltpu.VMEM((BBLK, D), jnp.float32),
        pltpu.VMEM((BBLK, D), jnp.float32),
        pltpu.VMEM((BBLK, D), jnp.float32),
        pltpu.VMEM((D // 8, 8, BBLK), jnp.float32),
        pltpu.VMEM((D // 8, 8, BBLK), jnp.float32),
        pltpu.VMEM((D // 8, 8, BBLK), jnp.float32),
        pltpu.VMEM((D // 8, 8, BBLK), jnp.float32),
        pltpu.SemaphoreType.DMA,
        pltpu.SemaphoreType.DMA,
        pltpu.SemaphoreType.DMA,
        pltpu.SemaphoreType.DMA,
        pltpu.SemaphoreType.DMA,
        pltpu.SemaphoreType.DMA,
        pltpu.SemaphoreType.DMA,
        pltpu.SemaphoreType.DMA,
    ],
    compiler_params=pltpu.CompilerParams(use_tc_tiling_on_sc=False,
                                        needs_layout_passes=False),
)
def _encoder(ids_units, table, pos, out,
             idx_v, pos_v, b0, b1, b2, b3, t0, t1, t2, t3,
             g0, g1, g2, g3, s0, s1, s2, s3):
    _IOTA16 = lax.iota(jnp.int32, 16)
    bufs = (b0, b1, b2, b3)
    stgs = (t0, t1, t2, t3)
    gsems = (g0, g1, g2, g3)
    ssems = (s0, s1, s2, s3)

    cid = lax.axis_index("c")
    sid = lax.axis_index("s")
    base_u = sid * PER_PAIR + lax.mul(cid, Q0)
    pltpu.sync_copy(ids_units.at[pl.ds(base_u, QMAX)], idx_v)
    pltpu.sync_copy(pos.at[pl.ds(0, S)], pos_v)

    def unit_coords(u):
        ug = base_u + u
        st = lax.shift_right_logical(ug, 6)
        bt = lax.rem(lax.shift_right_logical(ug, 3), 8)
        s8 = lax.rem(ug, 8)
        return st * 8 + s8, bt

    def gather(slot, u):
        return pltpu.make_async_copy(table.at[idx_v.at[u]], bufs[slot],
                                     gsems[slot])

    def store(slot, u):
        s, bt = unit_coords(u)
        return pltpu.make_async_copy(stgs[slot], out.at[s].at[:, bt],
                                     ssems[slot])

    def lane_min(v):
        acc = v
        for shift in (8, 4, 2, 1):
            g = lax.gather(
                acc, lax.rem(_IOTA16 + shift, 16)[:, None],
                dimension_numbers=lax.GatherDimensionNumbers(
                    offset_dims=(), collapsed_slice_dims=(0,),
                    start_index_map=(0,)),
                slice_sizes=(1,), mode=lax.GatherScatterMode.PROMISE_IN_BOUNDS)
            acc = jnp.minimum(acc, g)
        return acc[0]

    def compute(slot, u, has_pad):
        buf = bufs[slot]
        stg = stgs[slot]
        s, _ = unit_coords(u)

        @pl.when(has_pad)
        def _():
            for grp in range(BBLK // 16):
                idv = idx_v[u, pl.ds(grp * 16, 16)]
                @pl.when(lane_min(idv) == 0)
                def _():
                    zero = jnp.zeros((16,), jnp.float32)
                    for j in range(16):
                        @pl.when(idv[j] == 0)
                        def _():
                            for k in range(4):
                                buf[grp * 16 + j, pl.ds(k * 16, 16)] = zero

        def _dim(dhi, carry):
            pblk = lax.mul(lax.shift_right_logical(dhi, 1), 16)
            p = pos_v[s, pl.ds(pblk, 16)]
            lane0 = lax.mul(lax.rem(dhi, 2), 8)
            d0 = lax.mul(dhi, 8)
            for dlo in range(8):
                splat = lax.gather(
                    p, jnp.broadcast_to(lane0 + dlo, (16,))[:, None],
                    dimension_numbers=lax.GatherDimensionNumbers(
                        offset_dims=(), collapsed_slice_dims=(0,),
                        start_index_map=(0,)),
                    slice_sizes=(1,),
                    mode=lax.GatherScatterMode.PROMISE_IN_BOUNDS)
                cols = jnp.broadcast_to(d0 + dlo, (16,))
                for grp in range(BBLK // 16):
                    rows = _IOTA16 + (grp * 16)
                    vals = plsc.load_gather(buf, [rows, cols])
                    stg[dhi, dlo, pl.ds(grp * 16, 16)] = vals + splat
            return carry
        lax.fori_loop(0, D // 8, _dim, 0)

    def run(n):
        def _mn(t, acc):
            u = lax.shift_right_logical(t, 3)
            grp = lax.rem(t, 8)
            return jnp.minimum(acc, idx_v[u, pl.ds(grp * 16, 16)])

        acc = lax.fori_loop(0, lax.mul(n, BBLK // 16), _mn,
                            jnp.full((16,), jnp.iinfo(jnp.int32).max,
                                     jnp.int32))
        has_pad = lane_min(acc) == 0

        for u0 in range(NBUF - 1):
            gather(u0, jnp.int32(u0)).start()

        iters = lax.div(n, NBUF)

        def _iter(i, carry):
            for j in range(NBUF):
                u = NBUF * i + j
                nxt = u + NBUF - 1
                tgt = (j + NBUF - 1) % NBUF

                if j == 0:
                    @pl.when(i > 0)
                    def _():
                        store(tgt, nxt - NBUF).wait()
                    gather(tgt, nxt).start()
                else:
                    @pl.when(i < iters - 1)
                    def _():
                        store(tgt, nxt - NBUF).wait()
                        gather(tgt, nxt).start()

                gather(j, u).wait()
                compute(j, u, has_pad)
                store(j, u).start()
            return carry

        lax.fori_loop(0, iters, _iter, 0)

        for j in range(NBUF):
            store(j, n - NBUF + j).wait()

    run(jnp.where(cid == 0, jnp.int32(Q0), jnp.int32(Q1)))


def kernel(input_ids, token_table, pos_table):
    ids = input_ids.astype(jnp.int32)
    ids_units = (ids.reshape(B // BBLK, BBLK, S // 8, 8)
                 .transpose(2, 0, 3, 1)
                 .reshape(UNITS, BBLK))
    out5 = _encoder(ids_units, token_table, pos_table)
    return out5.transpose(2, 4, 0, 1, 3).reshape(B, S, D)


# speedup vs baseline: 1.2326x; 1.2326x over previous
import functools

import jax
import jax.numpy as jnp
from jax import lax
from jax.experimental import pallas as pl
from jax.experimental.pallas import tpu as pltpu
from jax.experimental.pallas import tpu_sc as plsc

VOCAB = 100000
D = 64
B, S = 1024, 200
BBLK = 128
UNITS = S * (B // BBLK)
PAIRS = 16
PER_PAIR = UNITS // PAIRS
Q0 = 24
Q1 = PER_PAIR - Q0
QMAX = max(Q0, Q1)
NBUF = 4

_mesh = plsc.VectorSubcoreMesh(core_axis_name="c", subcore_axis_name="s")


@functools.partial(
    pl.kernel,
    mesh=_mesh,
    out_type=jax.ShapeDtypeStruct((S, D // 8, B // BBLK, 8, BBLK), jnp.float32),
    scratch_types=[
        pltpu.VMEM((QMAX, BBLK), jnp.int32),
        pltpu.VMEM((S, D), jnp.float32),
        pltpu.VMEM((BBLK, D), jnp.float32),
        pltpu.VMEM((BBLK, D), jnp.float32),
        pltpu.VMEM((BBLK, D), jnp.float32),
        pltpu.VMEM((BBLK, D), jnp.float32),
        pltpu.VMEM((BBLK, 72), jnp.float32),
        pltpu.VMEM((BBLK, 72), jnp.float32),
        pltpu.VMEM((BBLK, 72), jnp.float32),
        pltpu.VMEM((BBLK, 72), jnp.float32),
        pltpu.VMEM((D // 8, 8, BBLK), jnp.float32),
        pltpu.VMEM((D // 8, 8, BBLK), jnp.float32),
        pltpu.VMEM((D // 8, 8, BBLK), jnp.float32),
        pltpu.VMEM((D // 8, 8, BBLK), jnp.float32),
        pltpu.SemaphoreType.DMA,
        pltpu.SemaphoreType.DMA,
        pltpu.SemaphoreType.DMA,
        pltpu.SemaphoreType.DMA,
        pltpu.SemaphoreType.DMA,
        pltpu.SemaphoreType.DMA,
        pltpu.SemaphoreType.DMA,
        pltpu.SemaphoreType.DMA,
    ],
    compiler_params=pltpu.CompilerParams(use_tc_tiling_on_sc=False,
                                        needs_layout_passes=False),
)
def _encoder(ids_units, table, pos, out,
             idx_v, pos_v, b0, b1, b2, b3, p0, p1, p2, p3, t0, t1, t2, t3,
             g0, g1, g2, g3, s0, s1, s2, s3):
    _IOTA16 = lax.iota(jnp.int32, 16)
    bufs = (b0, b1, b2, b3)
    pads = (p0, p1, p2, p3)
    stgs = (t0, t1, t2, t3)
    gsems = (g0, g1, g2, g3)
    ssems = (s0, s1, s2, s3)

    cid = lax.axis_index("c")
    sid = lax.axis_index("s")
    base_u = sid * PER_PAIR + lax.mul(cid, Q0)
    pltpu.sync_copy(ids_units.at[pl.ds(base_u, QMAX)], idx_v)
    pltpu.sync_copy(pos.at[pl.ds(0, S)], pos_v)

    def unit_coords(u):
        ug = base_u + u
        st = lax.shift_right_logical(ug, 6)
        bt = lax.rem(lax.shift_right_logical(ug, 3), 8)
        s8 = lax.rem(ug, 8)
        return st * 8 + s8, bt

    def gather(slot, u):
        return pltpu.make_async_copy(table.at[idx_v.at[u]], bufs[slot],
                                     gsems[slot])

    def store(slot, u):
        s, bt = unit_coords(u)
        return pltpu.make_async_copy(stgs[slot], out.at[s].at[:, bt],
                                     ssems[slot])

    def lane_min(v):
        acc = v
        for shift in (8, 4, 2, 1):
            g = lax.gather(
                acc, lax.rem(_IOTA16 + shift, 16)[:, None],
                dimension_numbers=lax.GatherDimensionNumbers(
                    offset_dims=(), collapsed_slice_dims=(0,),
                    start_index_map=(0,)),
                slice_sizes=(1,), mode=lax.GatherScatterMode.PROMISE_IN_BOUNDS)
            acc = jnp.minimum(acc, g)
        return acc[0]

    def compute(slot, u, has_pad):
        buf = bufs[slot]
        stg = stgs[slot]
        s, _ = unit_coords(u)

        @pl.when(has_pad)
        def _():
            for grp in range(BBLK // 16):
                idv = idx_v[u, pl.ds(grp * 16, 16)]
                @pl.when(lane_min(idv) == 0)
                def _():
                    zero = jnp.zeros((16,), jnp.float32)
                    for j in range(16):
                        @pl.when(idv[j] == 0)
                        def _():
                            for k in range(4):
                                buf[grp * 16 + j, pl.ds(k * 16, 16)] = zero

        pad = pads[slot]
        prow = [pos_v[s, pl.ds(k * 16, 16)] for k in range(4)]

        def _cp(i2, carry):
            for r2 in range(2):
                for k in range(4):
                    sl = pl.ds(k * 16, 16)
                    pad[i2 * 2 + r2, sl] = buf[i2 * 2 + r2, sl] + prow[k]
            return carry
        lax.fori_loop(0, BBLK // 2, _cp, 0)

        def _dim(dhi, carry):
            d0 = lax.mul(dhi, 8)
            for dlo in range(8):
                cols = jnp.broadcast_to(d0 + dlo, (16,))
                for grp in range(BBLK // 16):
                    rows = _IOTA16 + (grp * 16)
                    vals = plsc.load_gather(pad, [rows, cols])
                    stg[dhi, dlo, pl.ds(grp * 16, 16)] = vals
            return carry
        lax.fori_loop(0, D // 8, _dim, 0)

    def run(n):
        def _mn(t, acc):
            u = lax.shift_right_logical(t, 3)
            grp = lax.rem(t, 8)
            return jnp.minimum(acc, idx_v[u, pl.ds(grp * 16, 16)])

        acc = lax.fori_loop(0, lax.mul(n, BBLK // 16), _mn,
                            jnp.full((16,), jnp.iinfo(jnp.int32).max,
                                     jnp.int32))
        has_pad = lane_min(acc) == 0

        for u0 in range(NBUF - 1):
            gather(u0, jnp.int32(u0)).start()

        iters = lax.div(n, NBUF)

        def _iter(i, carry):
            for j in range(NBUF):
                u = NBUF * i + j
                nxt = u + NBUF - 1
                tgt = (j + NBUF - 1) % NBUF

                if j == 0:
                    @pl.when(i > 0)
                    def _():
                        store(tgt, nxt - NBUF).wait()
                    gather(tgt, nxt).start()
                else:
                    @pl.when(i < iters - 1)
                    def _():
                        store(tgt, nxt - NBUF).wait()
                        gather(tgt, nxt).start()

                gather(j, u).wait()
                compute(j, u, has_pad)
                store(j, u).start()
            return carry

        lax.fori_loop(0, iters, _iter, 0)

        for j in range(NBUF):
            store(j, n - NBUF + j).wait()

    run(jnp.where(cid == 0, jnp.int32(Q0), jnp.int32(Q1)))


def kernel(input_ids, token_table, pos_table):
    ids = input_ids.astype(jnp.int32)
    ids_units = (ids.reshape(B // BBLK, BBLK, S // 8, 8)
                 .transpose(2, 0, 3, 1)
                 .reshape(UNITS, BBLK))
    out5 = _encoder(ids_units, token_table, pos_table)
    return out5.transpose(2, 4, 0, 1, 3).reshape(B, S, D)


# speedup vs baseline: 2.3301x; 1.8904x over previous
import functools

import jax
import jax.numpy as jnp
from jax import lax
from jax.experimental import pallas as pl
from jax.experimental.pallas import tpu as pltpu
from jax.experimental.pallas import tpu_sc as plsc

VOCAB = 100000
D = 64
B, S = 1024, 200
NW = 32
PAIR_SEQ = B // 16
Q0 = 16
Q1 = PAIR_SEQ - Q0
QMAX = max(Q0, Q1)
HALF = 100
NBUF = 4

_mesh = plsc.VectorSubcoreMesh(core_axis_name="c", subcore_axis_name="s")


@functools.partial(
    pl.kernel,
    mesh=_mesh,
    out_type=jax.ShapeDtypeStruct((B, S, D), jnp.float32),
    scratch_types=[
        pltpu.VMEM((QMAX * 2, HALF), jnp.int32),
        pltpu.VMEM((QMAX * S + 16,), jnp.int32),
        pltpu.VMEM((S, D), jnp.float32),
        pltpu.VMEM((S, D), jnp.float32),
        pltpu.VMEM((S, D), jnp.float32),
        pltpu.VMEM((S, D), jnp.float32),
        pltpu.VMEM((S, D), jnp.float32),
        pltpu.SemaphoreType.DMA,
        pltpu.SemaphoreType.DMA,
        pltpu.SemaphoreType.DMA,
        pltpu.SemaphoreType.DMA,
        pltpu.SemaphoreType.DMA,
        pltpu.SemaphoreType.DMA,
        pltpu.SemaphoreType.DMA,
        pltpu.SemaphoreType.DMA,
    ],
    compiler_params=pltpu.CompilerParams(use_tc_tiling_on_sc=False),
)
def _encoder(ids_stream, ids_chk, table, pos, out,
             idx_v, chk_v, pos_v, b0, b1, b2, b3,
             g0, g1, g2, g3, s0, s1, s2, s3):
    bufs = (b0, b1, b2, b3)
    gsems = (g0, g1, g2, g3)
    ssems = (s0, s1, s2, s3)

    cid = lax.axis_index("c")
    sid = lax.axis_index("s")
    base = sid * PAIR_SEQ + lax.mul(cid, Q0)
    pltpu.sync_copy(ids_stream.at[pl.ds(2 * base, 2 * QMAX)], idx_v)
    pltpu.sync_copy(ids_chk.at[pl.ds(base * S, QMAX * S)],
                    chk_v.at[pl.ds(0, QMAX * S)])
    pltpu.sync_copy(pos.at[pl.ds(0, S)], pos_v)

    def gathers(slot, c):
        return (pltpu.make_async_copy(table.at[idx_v.at[2 * c]],
                                      bufs[slot].at[pl.ds(0, HALF)],
                                      gsems[slot]),
                pltpu.make_async_copy(table.at[idx_v.at[2 * c + 1]],
                                      bufs[slot].at[pl.ds(HALF, HALF)],
                                      gsems[slot]))

    def store(slot, c):
        return pltpu.make_async_copy(bufs[slot], out.at[base + c], ssems[slot])

    def compute(slot, c, has_pad):
        buf = bufs[slot]
        tok0 = c * S

        @pl.when(has_pad)
        def _():
            def _fix(r, c2):
                idv = chk_v[pl.ds(tok0 + r, 16)]
                @pl.when(idv[0] == 0)
                def _():
                    zero = jnp.zeros((16,), jnp.float32)
                    for k in range(4):
                        buf[r, pl.ds(k * 16, 16)] = zero
                return c2
            lax.fori_loop(0, S, _fix, 0)

        def _add(r, c2):
            for k in range(4):
                sl = pl.ds(k * 16, 16)
                buf[r, sl] = buf[r, sl] + pos_v[r, sl]
            return c2
        lax.fori_loop(0, S, _add, 0)

    def run(n):
        def _mn(i, acc):
            return jnp.minimum(acc, chk_v[pl.ds(i * 16, 16)])

        acc = lax.fori_loop(0, n * S // 16, _mn,
                            jnp.full((16,), jnp.iinfo(jnp.int32).max, jnp.int32))
        lanes = lax.iota(jnp.int32, 16)
        for shift in (8, 4, 2, 1):
            g = lax.gather(
                acc, lax.rem(lanes + shift, 16)[:, None],
                dimension_numbers=lax.GatherDimensionNumbers(
                    offset_dims=(), collapsed_slice_dims=(0,),
                    start_index_map=(0,)),
                slice_sizes=(1,), mode=lax.GatherScatterMode.PROMISE_IN_BOUNDS)
            acc = jnp.minimum(acc, g)
        has_pad = acc[0] == 0

        for c0 in range(NBUF - 1):
            a, b = gathers(c0, jnp.int32(c0))
            a.start()
            b.start()

        iters = n // NBUF

        def _iter(i, carry):
            for j in range(NBUF):
                c = NBUF * i + j
                nxt = c + NBUF - 1
                tgt = (j + NBUF - 1) % NBUF

                def _prefetch():
                    a, b = gathers(tgt, nxt)
                    a.start()
                    b.start()

                if j == 0:
                    @pl.when(i > 0)
                    def _():
                        store(tgt, nxt - NBUF).wait()
                    _prefetch()
                else:
                    @pl.when(i < iters - 1)
                    def _():
                        store(tgt, nxt - NBUF).wait()
                        _prefetch()

                ga, gb = gathers(j, c)
                ga.wait()
                gb.wait()
                compute(j, c, has_pad)
                store(j, c).start()
            return carry

        lax.fori_loop(0, iters, _iter, 0)

        for j in range(NBUF):
            store(j, jnp.int32(n - NBUF + j)).wait()

    @pl.when(cid == 0)
    def _():
        run(Q0)

    @pl.when(cid == 1)
    def _():
        run(Q1)


def kernel(input_ids, token_table, pos_table):
    ids = input_ids.astype(jnp.int32)
    ids_stream = ids.reshape(B * 2, HALF)
    ids_chk = ids.reshape(B * S)
    return _encoder(ids_stream, ids_chk, token_table, pos_table)
